# baseline (speedup 1.0000x reference)
import functools
import math

import jax
import jax.numpy as jnp
from jax import lax
from jax.experimental import pallas as pl
from jax.experimental.pallas import tpu as pltpu
from jax.experimental.pallas import tpu_sc as plsc

NC = 2
NS = 16
L = 16
NW = NC * NS

V = 1000000
R = 4096
S = 200
D = 64
JB = S // 8
IB = R // 128
NG = 4
NGROUP = S // NG
SCALE = math.sqrt(D)

TBW = 1024
TGRID = -(-V // TBW)
V2 = TGRID * TBW // 2

_mesh = plsc.VectorSubcoreMesh(core_axis_name="c", subcore_axis_name="s")


def _eye(scale):
    a = lax.broadcasted_iota(jnp.int32, (D, D), 0)
    b = lax.broadcasted_iota(jnp.int32, (D, D), 1)
    return jnp.where(a == b, jnp.float32(scale), jnp.float32(0.0))


def _tbl_body(i_ref, o_ref):
    t = lax.dot_general(
        i_ref[...], _eye(SCALE), (((0,), (0,)), ((), ())),
        preferred_element_type=jnp.float32,
    )
    o_ref[...] = jnp.concatenate([t[: TBW // 2], t[TBW // 2 :]], axis=1)


_tc_table = pl.pallas_call(
    _tbl_body,
    grid=(TGRID,),
    in_specs=[pl.BlockSpec((D, TBW), lambda g: (0, g))],
    out_specs=pl.BlockSpec((TBW // 2, 2 * D), lambda g: (g, 0)),
    out_shape=jax.ShapeDtypeStruct((V2, 2 * D), jnp.float32),
)


@functools.partial(
    pl.kernel,
    out_type=jax.ShapeDtypeStruct((S, IB, D, 2 * D), jnp.float32),
    mesh=_mesh,
    scratch_types=[
        pltpu.VMEM((JB, 8, 128), jnp.int32),
        pltpu.VMEM((NG, 128, D), jnp.float32),
        pltpu.VMEM((NG, D, 2 * D), jnp.float32),
    ]
    + [pltpu.SemaphoreType.DMA] * (2 * NG),
    compiler_params=pltpu.CompilerParams(
        use_tc_tiling_on_sc=False, needs_layout_passes=False
    ),
)
def _embed(xt_hbm, table_hbm, out_hbm, idx_v, g_v, o_v, *sems):
    gsem, osem = sems[:NG], sems[NG:]
    wid = lax.axis_index("s") * NC + lax.axis_index("c")

    def stage(jb, carry):
        pltpu.sync_copy(xt_hbm.at[jb, wid], idx_v.at[jb])
        for r8 in range(8):
            for c in range(8):
                sl = pl.ds(c * L, L)
                v = idx_v[jb, r8, sl]
                idx_v[jb, r8, sl] = (
                    (v & ~1023) + ((v & 511) << 1) + ((v >> 9) & 1)
                )
        return carry

    lax.fori_loop(0, JB, stage, 0)

    def start_gather(b, j):
        pltpu.async_copy(
            table_hbm.at[idx_v.at[j // 8, j % 8]], g_v.at[b], gsem[b]
        )

    for b in range(NG):
        start_gather(b, b)

    def group(g, carry):
        j0 = NG * g
        for b in range(NG):
            j = j0 + b
            pltpu.make_async_copy(
                table_hbm.at[idx_v.at[0, 0]], g_v.at[b], gsem[b]
            ).wait()

            @pl.when(g > 0)
            def _():
                pltpu.make_async_copy(
                    o_v.at[b], out_hbm.at[0, 0], osem[b]
                ).wait()

            def srow(k, c2, b=b):
                r0 = k * 8
                for dr in range(8):
                    for q in range(8):
                        src = r0 + dr + (D if q >= 4 else 0)
                        v = g_v[b, src, pl.ds((q % 4) * L, L)]
                        o_v[b, r0 + dr, pl.ds(q * L, L)] = v
                return c2

            lax.fori_loop(0, 8, srow, 0)

            pltpu.async_copy(o_v.at[b], out_hbm.at[j, wid], osem[b])

            @pl.when(g < NGROUP - 1)
            def _():
                start_gather(b, j + NG)
        return carry

    lax.fori_loop(0, NGROUP, group, 0)

    for b in range(NG):
        pltpu.make_async_copy(o_v.at[b], out_hbm.at[0, 0], osem[b]).wait()


def _out_body(i_ref, o_ref):
    t3 = lax.dot_general(
        i_ref[:, 0], _eye(1.0), (((1,), (0,)), ((), ())),
        preferred_element_type=jnp.float32,
    )
    o = jnp.concatenate([t3[:, :D, :], t3[:, D:, :]], axis=2)
    o_ref[:, :, 0] = o.reshape(8, 8, 8, 2 * D)


_tc_out = pl.pallas_call(
    _out_body,
    grid=(JB, IB),
    in_specs=[pl.BlockSpec((8, 1, D, 2 * D), lambda jb, ib: (jb, ib, 0, 0))],
    out_specs=pl.BlockSpec(
        (8, 8, 1, 8, 2 * D), lambda jb, ib: (jb, 0, ib, 0, 0)
    ),
    out_shape=jax.ShapeDtypeStruct((S, 8, IB, 8, 2 * D), jnp.float32),
)


def kernel(x, table):
    xt = x.T.reshape(JB, 8, IB, 128).transpose(0, 2, 1, 3)
    tbl2 = _tc_table(table.T)
    out_sc = _embed(xt, tbl2.reshape(2 * V2, D))
    out5 = _tc_out(out_sc)
    return out5.transpose(2, 4, 0, 1, 3).reshape(R, S, D)


# speedup vs baseline: 1.5744x; 1.5744x over previous
import functools
import math

import jax
import jax.numpy as jnp
from jax import lax
from jax.experimental import pallas as pl
from jax.experimental.pallas import tpu as pltpu
from jax.experimental.pallas import tpu_sc as plsc

NC = 2
NS = 16
L = 16
NW = NC * NS

V = 1000000
R = 4096
S = 200
D = 64
JB = S // 8
IB = R // 128
NG = 4
NGROUP = S // NG
SCALE = math.sqrt(D)

TBW = 2048
TGRID = -(-V // TBW)
V2 = TGRID * TBW // 2

_mesh = plsc.VectorSubcoreMesh(core_axis_name="c", subcore_axis_name="s")


def _eye(scale):
    a = lax.broadcasted_iota(jnp.int32, (D, D), 0)
    b = lax.broadcasted_iota(jnp.int32, (D, D), 1)
    return jnp.where(a == b, jnp.float32(scale), jnp.float32(0.0))


def _tbl_body(i_ref, o_ref):
    t = lax.dot_general(
        i_ref[...], _eye(SCALE), (((0,), (0,)), ((), ())),
        preferred_element_type=jnp.float32,
    )
    o_ref[...] = jnp.concatenate([t[: TBW // 2], t[TBW // 2 :]], axis=1)


_tc_table = pl.pallas_call(
    _tbl_body,
    grid=(TGRID,),
    in_specs=[pl.BlockSpec((D, TBW), lambda g: (0, g))],
    out_specs=pl.BlockSpec((TBW // 2, 2 * D), lambda g: (g, 0)),
    out_shape=jax.ShapeDtypeStruct((V2, 2 * D), jnp.float32),
)


@functools.partial(
    pl.kernel,
    out_type=jax.ShapeDtypeStruct((S, IB, D, 2 * D), jnp.float32),
    mesh=_mesh,
    scratch_types=[
        pltpu.VMEM((JB, 8, 128), jnp.int32),
        pltpu.VMEM((NG, 128, D), jnp.float32),
        pltpu.VMEM((NG, D, 2 * D), jnp.float32),
    ]
    + [pltpu.SemaphoreType.DMA] * (2 * NG),
    compiler_params=pltpu.CompilerParams(
        use_tc_tiling_on_sc=False, needs_layout_passes=False
    ),
)
def _embed(xt_hbm, table_hbm, out_hbm, idx_v, g_v, o_v, *sems):
    gsem, osem = sems[:NG], sems[NG:]
    wid = lax.axis_index("s") * NC + lax.axis_index("c")

    def stage(jb, carry):
        pltpu.sync_copy(xt_hbm.at[jb, wid], idx_v.at[jb])
        half = TBW // 2
        sh = half.bit_length() - 1
        for r8 in range(8):
            for c in range(8):
                sl = pl.ds(c * L, L)
                v = idx_v[jb, r8, sl]
                idx_v[jb, r8, sl] = (
                    (v & ~(TBW - 1)) + ((v & (half - 1)) << 1) + ((v >> sh) & 1)
                )
        return carry

    lax.fori_loop(0, JB, stage, 0)

    def start_gather(b, j):
        pltpu.async_copy(
            table_hbm.at[idx_v.at[j // 8, j % 8]], g_v.at[b], gsem[b]
        )

    for b in range(NG):
        start_gather(b, b)

    def group(g, carry):
        j0 = NG * g
        for b in range(NG):
            j = j0 + b
            pltpu.make_async_copy(
                table_hbm.at[idx_v.at[0, 0]], g_v.at[b], gsem[b]
            ).wait()

            @pl.when(g > 0)
            def _():
                pltpu.make_async_copy(
                    o_v.at[b], out_hbm.at[0, 0], osem[b]
                ).wait()

            def srow(k, c2, b=b):
                r0 = k * 8
                for dr in range(8):
                    for q in range(8):
                        src = r0 + dr + (D if q >= 4 else 0)
                        v = g_v[b, src, pl.ds((q % 4) * L, L)]
                        o_v[b, r0 + dr, pl.ds(q * L, L)] = v
                return c2

            lax.fori_loop(0, 8, srow, 0)

            pltpu.async_copy(o_v.at[b], out_hbm.at[j, wid], osem[b])

            @pl.when(g < NGROUP - 1)
            def _():
                start_gather(b, j + NG)
        return carry

    lax.fori_loop(0, NGROUP, group, 0)

    for b in range(NG):
        pltpu.make_async_copy(o_v.at[b], out_hbm.at[0, 0], osem[b]).wait()


JBLK = 40


def _out_body(i_ref, o_ref):
    t3 = lax.dot_general(
        i_ref[:, 0], _eye(1.0), (((1,), (0,)), ((), ())),
        preferred_element_type=jnp.float32,
    )
    o = jnp.concatenate([t3[:, :D, :], t3[:, D:, :]], axis=2)
    o_ref[:, :, 0] = o.reshape(JBLK, 8, 8, 2 * D)


_tc_out = pl.pallas_call(
    _out_body,
    grid=(S // JBLK, IB),
    in_specs=[
        pl.BlockSpec((JBLK, 1, D, 2 * D), lambda jb, ib: (jb, ib, 0, 0))
    ],
    out_specs=pl.BlockSpec(
        (JBLK, 8, 1, 8, 2 * D), lambda jb, ib: (jb, 0, ib, 0, 0)
    ),
    out_shape=jax.ShapeDtypeStruct((S, 8, IB, 8, 2 * D), jnp.float32),
)


def kernel(x, table):
    xt = x.T.reshape(JB, 8, IB, 128).transpose(0, 2, 1, 3)
    tbl2 = _tc_table(table.T)
    out_sc = _embed(xt, tbl2.reshape(2 * V2, D))
    out5 = _tc_out(out_sc)
    return out5.transpose(2, 4, 0, 1, 3).reshape(R, S, D)


# speedup vs baseline: 1.8344x; 1.1651x over previous
import functools
import math

import jax
import jax.numpy as jnp
from jax import lax
from jax.experimental import pallas as pl
from jax.experimental.pallas import tpu as pltpu
from jax.experimental.pallas import tpu_sc as plsc

NC = 2
NS = 16
L = 16
NW = NC * NS

V = 1000000
R = 4096
S = 200
D = 64
JB = S // 8
IB = R // 128
NG = 4
NGROUP = S // NG
SCALE = math.sqrt(D)

TBW = 4096
TGRID = -(-V // TBW)
V2 = TGRID * TBW // 2

_mesh = plsc.VectorSubcoreMesh(core_axis_name="c", subcore_axis_name="s")


def _eye(scale):
    a = lax.broadcasted_iota(jnp.int32, (D, D), 0)
    b = lax.broadcasted_iota(jnp.int32, (D, D), 1)
    return jnp.where(a == b, jnp.float32(scale), jnp.float32(0.0))


def _tbl_body(i_ref, o_ref):
    t = lax.dot_general(
        i_ref[...], _eye(SCALE), (((0,), (0,)), ((), ())),
        preferred_element_type=jnp.float32,
    )
    o_ref[...] = jnp.concatenate([t[: TBW // 2], t[TBW // 2 :]], axis=1)


_tc_table = pl.pallas_call(
    _tbl_body,
    grid=(TGRID,),
    in_specs=[pl.BlockSpec((D, TBW), lambda g: (0, g))],
    out_specs=pl.BlockSpec((TBW // 2, 2 * D), lambda g: (g, 0)),
    out_shape=jax.ShapeDtypeStruct((V2, 2 * D), jnp.float32),
)


@functools.partial(
    pl.kernel,
    out_type=jax.ShapeDtypeStruct((S, IB, D, 2 * D), jnp.float32),
    mesh=_mesh,
    scratch_types=[
        pltpu.VMEM((JB, 8, 128), jnp.int32),
        pltpu.VMEM((NG, 128, D), jnp.float32),
        pltpu.VMEM((NG, D, 2 * D), jnp.float32),
    ]
    + [pltpu.SemaphoreType.DMA] * (2 * NG),
    compiler_params=pltpu.CompilerParams(
        use_tc_tiling_on_sc=False, needs_layout_passes=False
    ),
)
def _embed(xt_hbm, table_hbm, out_hbm, idx_v, g_v, o_v, *sems):
    gsem, osem = sems[:NG], sems[NG:]
    wid = lax.axis_index("s") * NC + lax.axis_index("c")

    def stage(jb, carry):
        pltpu.sync_copy(xt_hbm.at[jb, wid], idx_v.at[jb])
        half = TBW // 2
        sh = half.bit_length() - 1
        for r8 in range(8):
            for c in range(8):
                sl = pl.ds(c * L, L)
                v = idx_v[jb, r8, sl]
                idx_v[jb, r8, sl] = (
                    (v & ~(TBW - 1)) + ((v & (half - 1)) << 1) + ((v >> sh) & 1)
                )
        return carry

    lax.fori_loop(0, JB, stage, 0)

    def start_gather(b, j):
        pltpu.async_copy(
            table_hbm.at[idx_v.at[j // 8, j % 8]], g_v.at[b], gsem[b]
        )

    for b in range(NG):
        start_gather(b, b)

    def group(g, carry):
        j0 = NG * g
        for b in range(NG):
            j = j0 + b
            pltpu.make_async_copy(
                table_hbm.at[idx_v.at[0, 0]], g_v.at[b], gsem[b]
            ).wait()

            @pl.when(g > 0)
            def _():
                pltpu.make_async_copy(
                    o_v.at[b], out_hbm.at[0, 0], osem[b]
                ).wait()

            def srow(k, c2, b=b):
                r0 = k * 8
                for dr in range(8):
                    for q in range(8):
                        src = r0 + dr + (D if q >= 4 else 0)
                        v = g_v[b, src, pl.ds((q % 4) * L, L)]
                        o_v[b, r0 + dr, pl.ds(q * L, L)] = v
                return c2

            lax.fori_loop(0, 8, srow, 0)

            pltpu.async_copy(o_v.at[b], out_hbm.at[j, wid], osem[b])

            @pl.when(g < NGROUP - 1)
            def _():
                start_gather(b, j + NG)
        return carry

    lax.fori_loop(0, NGROUP, group, 0)

    for b in range(NG):
        pltpu.make_async_copy(o_v.at[b], out_hbm.at[0, 0], osem[b]).wait()


JBLK = 50


def _out_body(i_ref, o_ref):
    t3 = lax.dot_general(
        i_ref[:, 0], _eye(1.0), (((1,), (0,)), ((), ())),
        preferred_element_type=jnp.float32,
    )
    o = jnp.concatenate([t3[:, :D, :], t3[:, D:, :]], axis=2)
    o_ref[:, :, 0] = o.reshape(JBLK, 8, 8, 2 * D)


_tc_out = pl.pallas_call(
    _out_body,
    grid=(S // JBLK, IB),
    in_specs=[
        pl.BlockSpec((JBLK, 1, D, 2 * D), lambda jb, ib: (jb, ib, 0, 0))
    ],
    out_specs=pl.BlockSpec(
        (JBLK, 8, 1, 8, 2 * D), lambda jb, ib: (jb, 0, ib, 0, 0)
    ),
    out_shape=jax.ShapeDtypeStruct((S, 8, IB, 8, 2 * D), jnp.float32),
)


def kernel(x, table):
    xt = x.T.reshape(JB, 8, IB, 128).transpose(0, 2, 1, 3)
    tbl2 = _tc_table(table.T)
    out_sc = _embed(xt, tbl2.reshape(2 * V2, D))
    out5 = _tc_out(out_sc)
    return out5.transpose(2, 4, 0, 1, 3).reshape(R, S, D)
